# baseline (speedup 1.0000x reference)
import functools

import jax
import jax.numpy as jnp
from jax import lax
from jax.experimental import pallas as pl
from jax.experimental.pallas import tpu as pltpu
from jax.experimental.pallas import tpu_sc as plsc

N_EDGES = 1600000
N_SEG = 512
NUM_CORES = 2
NUM_SUBCORES = 16
LANES = 16
NW = NUM_CORES * NUM_SUBCORES
EPW = N_EDGES // NW
NCHUNK = 25
CSZ = EPW // NCHUNK
CVECS = CSZ // LANES
GRP = 5
UNROLL = 5
WAVE = 8


def _sc_partials(x, idx):
    mesh = plsc.VectorSubcoreMesh(
        core_axis_name="c", subcore_axis_name="s")

    @functools.partial(
        pl.kernel,
        out_type=[
            jax.ShapeDtypeStruct((NW, N_SEG), jnp.float32),
            jax.ShapeDtypeStruct((NW, N_SEG), jnp.float32),
        ],
        mesh=mesh,
        compiler_params=pltpu.CompilerParams(
            needs_layout_passes=False,
            disable_bounds_checks=True,
        ),
        scratch_types=[
            pltpu.VMEM((EPW,), jnp.float32),
            pltpu.VMEM((EPW,), jnp.int32),
            pltpu.VMEM((N_SEG,), jnp.float32),
            pltpu.VMEM((N_SEG,), jnp.float32),
            pltpu.VMEM((N_SEG,), jnp.float32),
            pltpu.VMEM((N_SEG,), jnp.float32),
            pltpu.SemaphoreType.DMA,
            pltpu.SemaphoreType.DMA,
        ],
    )
    def k(x_hbm, idx_hbm, sums_hbm, counts_hbm,
          xv, iv, sums_a, counts_a, sums_b, counts_b, sem_x, sem_i):
        wid = lax.axis_index("s") * NUM_CORES + lax.axis_index("c")
        base = wid * EPW
        def fire(c):
            cpx = pltpu.make_async_copy(
                x_hbm.at[pl.ds(base + c * CSZ, CSZ)],
                xv.at[pl.ds(c * CSZ, CSZ)], sem_x)
            cpi = pltpu.make_async_copy(
                idx_hbm.at[pl.ds(base + c * CSZ, CSZ)],
                iv.at[pl.ds(c * CSZ, CSZ)], sem_i)
            cpx.start()
            cpi.start()
            return (cpx, cpi)

        cps = [fire(c) for c in range(WAVE)]
        zero = jnp.zeros((LANES,), jnp.float32)
        sum_refs = (sums_a, sums_b)
        cnt_refs = (counts_a, counts_b)
        for j in range(N_SEG // LANES):
            sl = pl.ds(j * LANES, LANES)
            for r in sum_refs + cnt_refs:
                r[sl] = zero

        ones = jnp.ones((LANES,), jnp.float32)

        def make_body(cbase):
            def body(i, carry):
                off = pl.multiple_of(cbase + i * (GRP * LANES), LANES)
                ivs = [iv[pl.ds(off + q * LANES, LANES)] for q in range(GRP)]
                xvs = [xv[pl.ds(off + q * LANES, LANES)] for q in range(GRP)]
                for q in range(GRP):
                    plsc.addupdate_scatter(sum_refs[q % 2], [ivs[q]], xvs[q])
                    plsc.addupdate_scatter(cnt_refs[q % 2], [ivs[q]], ones)
                return carry
            return body

        for c in range(NCHUNK):
            cps[c][0].wait()
            cps[c][1].wait()
            if c + WAVE < NCHUNK:
                cps.append(fire(c + WAVE))
            cbase = c * CSZ
            lax.fori_loop(0, CVECS // GRP, make_body(cbase), 0, unroll=UNROLL)
            for t in range(CVECS % GRP):
                toff = cbase + ((CVECS // GRP) * GRP + t) * LANES
                it = iv[pl.ds(toff, LANES)]
                xt = xv[pl.ds(toff, LANES)]
                plsc.addupdate_scatter(sum_refs[t % 2], [it], xt)
                plsc.addupdate_scatter(cnt_refs[t % 2], [it], ones)

        for j in range(N_SEG // LANES):
            sl = pl.ds(j * LANES, LANES)
            sums_a[sl] = sums_a[sl] + sums_b[sl]
            counts_a[sl] = counts_a[sl] + counts_b[sl]

        pltpu.sync_copy(sums_a, sums_hbm.at[wid])
        pltpu.sync_copy(counts_a, counts_hbm.at[wid])

    return k(x, idx)


def _finalize(sums, counts, target2d):
    def body(s_ref, c_ref, t_ref, o_ref):
        s = jnp.sum(s_ref[...], axis=0, keepdims=True)
        c = jnp.sum(c_ref[...], axis=0, keepdims=True)
        mean = s / jnp.maximum(c, 1.0)
        d = mean - t_ref[...]
        o_ref[0, 0] = jnp.sum(d * d) * (1.0 / N_SEG)

    return pl.pallas_call(
        body,
        out_shape=jax.ShapeDtypeStruct((1, 1), jnp.float32),
        out_specs=pl.BlockSpec(memory_space=pltpu.SMEM),
    )(sums, counts, target2d)


def kernel(x, idx, target_mean_weights):
    if x.ndim > 1 and x.shape[1] == 1:
        x = jnp.squeeze(x, axis=1)
    sums, counts = _sc_partials(x, idx.astype(jnp.int32))
    out = _finalize(sums, counts, target_mean_weights.reshape(1, N_SEG))
    return out[0, 0]


# speedup vs baseline: 1.2355x; 1.2355x over previous
import functools

import jax
import jax.numpy as jnp
from jax import lax
from jax.experimental import pallas as pl
from jax.experimental.pallas import tpu as pltpu
from jax.experimental.pallas import tpu_sc as plsc

N_EDGES = 1600000
N_SEG = 512
NUM_CORES = 2
NUM_SUBCORES = 16
LANES = 16
NW = NUM_CORES * NUM_SUBCORES
EPW = N_EDGES // NW
NCHUNK = 5
CSZ = EPW // NCHUNK
CVECS = CSZ // LANES
GRP = 8
UNROLL = 2
WAVE = NCHUNK


def _sc_partials(x, idx):
    mesh = plsc.VectorSubcoreMesh(
        core_axis_name="c", subcore_axis_name="s")

    @functools.partial(
        pl.kernel,
        out_type=[
            jax.ShapeDtypeStruct((NW, N_SEG), jnp.float32),
            jax.ShapeDtypeStruct((NW, N_SEG), jnp.float32),
        ],
        mesh=mesh,
        compiler_params=pltpu.CompilerParams(
            needs_layout_passes=False,
            disable_bounds_checks=True,
        ),
        scratch_types=[
            pltpu.VMEM((EPW,), jnp.float32),
            pltpu.VMEM((EPW,), jnp.int32),
            pltpu.VMEM((N_SEG,), jnp.float32),
            pltpu.VMEM((N_SEG,), jnp.float32),
            pltpu.VMEM((N_SEG,), jnp.float32),
            pltpu.VMEM((N_SEG,), jnp.float32),
            pltpu.SemaphoreType.DMA,
            pltpu.SemaphoreType.DMA,
        ],
    )
    def k(x_hbm, idx_hbm, sums_hbm, counts_hbm,
          xv, iv, sums_a, counts_a, sums_b, counts_b, sem_x, sem_i):
        wid = lax.axis_index("s") * NUM_CORES + lax.axis_index("c")
        base = wid * EPW
        def fire(c):
            cpx = pltpu.make_async_copy(
                x_hbm.at[pl.ds(base + c * CSZ, CSZ)],
                xv.at[pl.ds(c * CSZ, CSZ)], sem_x)
            cpi = pltpu.make_async_copy(
                idx_hbm.at[pl.ds(base + c * CSZ, CSZ)],
                iv.at[pl.ds(c * CSZ, CSZ)], sem_i)
            cpx.start()
            cpi.start()
            return (cpx, cpi)

        cps = [fire(c) for c in range(WAVE)]
        zero = jnp.zeros((LANES,), jnp.float32)
        sum_refs = (sums_a, sums_b)
        cnt_refs = (counts_a, counts_b)
        for j in range(N_SEG // LANES):
            sl = pl.ds(j * LANES, LANES)
            for r in sum_refs + cnt_refs:
                r[sl] = zero

        ones = jnp.ones((LANES,), jnp.float32)

        def make_body(cbase):
            def body(i, carry):
                off = pl.multiple_of(cbase + i * (GRP * LANES), LANES)
                ivs = [iv[pl.ds(off + q * LANES, LANES)] for q in range(GRP)]
                xvs = [xv[pl.ds(off + q * LANES, LANES)] for q in range(GRP)]
                for q in range(GRP):
                    plsc.addupdate_scatter(sum_refs[q % 2], [ivs[q]], xvs[q])
                    plsc.addupdate_scatter(cnt_refs[q % 2], [ivs[q]], ones)
                return carry
            return body

        for c in range(NCHUNK):
            cps[c][0].wait()
            cps[c][1].wait()
            if c + WAVE < NCHUNK:
                cps.append(fire(c + WAVE))
            cbase = c * CSZ
            lax.fori_loop(0, CVECS // GRP, make_body(cbase), 0, unroll=UNROLL)
            for t in range(CVECS % GRP):
                toff = cbase + ((CVECS // GRP) * GRP + t) * LANES
                it = iv[pl.ds(toff, LANES)]
                xt = xv[pl.ds(toff, LANES)]
                plsc.addupdate_scatter(sum_refs[t % 2], [it], xt)
                plsc.addupdate_scatter(cnt_refs[t % 2], [it], ones)

        for j in range(N_SEG // LANES):
            sl = pl.ds(j * LANES, LANES)
            sums_a[sl] = sums_a[sl] + sums_b[sl]
            counts_a[sl] = counts_a[sl] + counts_b[sl]

        pltpu.sync_copy(sums_a, sums_hbm.at[wid])
        pltpu.sync_copy(counts_a, counts_hbm.at[wid])

    return k(x, idx)


def _finalize(sums, counts, target2d):
    def body(s_ref, c_ref, t_ref, o_ref):
        s = jnp.sum(s_ref[...], axis=0, keepdims=True)
        c = jnp.sum(c_ref[...], axis=0, keepdims=True)
        mean = s / jnp.maximum(c, 1.0)
        d = mean - t_ref[...]
        o_ref[0, 0] = jnp.sum(d * d) * (1.0 / N_SEG)

    return pl.pallas_call(
        body,
        out_shape=jax.ShapeDtypeStruct((1, 1), jnp.float32),
        out_specs=pl.BlockSpec(memory_space=pltpu.SMEM),
    )(sums, counts, target2d)


def kernel(x, idx, target_mean_weights):
    if x.ndim > 1 and x.shape[1] == 1:
        x = jnp.squeeze(x, axis=1)
    sums, counts = _sc_partials(x, idx.astype(jnp.int32))
    out = _finalize(sums, counts, target_mean_weights.reshape(1, N_SEG))
    return out[0, 0]
